# baseline (speedup 1.0000x reference)
import math

import jax
import jax.numpy as jnp
from jax.experimental import pallas as pl
from jax.experimental.pallas import tpu as pltpu

_NUM_HEADS = 16
_NUM_BUCKETS = 32
_MAX_DISTANCE = 128
_KEY_LEN = 2048
_QUERY_LEN = 2048


def _build_src_kernel(zero_ref, texp_ref, src_ref):
    w = jax.lax.broadcasted_iota(jnp.int32, (40, 2048), 0)
    c = jax.lax.broadcasted_iota(jnp.int32, (40, 2048), 1)
    i_row = w * 128 + (c >> 4)
    d = zero_ref[0, 0] + 2047 - i_row
    nb = _NUM_BUCKETS // 2
    ret = jnp.where(d > 0, nb, 0).astype(jnp.int32)
    ad = jnp.abs(d)
    max_exact = nb // 2
    is_small = ad < max_exact
    rp_f = jnp.maximum(ad, 1).astype(jnp.float32)
    val_large = max_exact + (
        jnp.log(rp_f / max_exact) / math.log(_MAX_DISTANCE / max_exact) * (nb - max_exact)
    ).astype(jnp.int32)
    val_large = jnp.minimum(val_large, nb - 1)
    bucket = ret + jnp.where(is_small, ad, val_large)
    acc = jnp.zeros((40, 2048), jnp.float32)
    for b in range(_NUM_BUCKETS):
        acc = acc + jnp.where(bucket == b, texp_ref[b, :][None, :], 0.0)
    src_ref[:, :] = acc


_APB = 8


def _expand_kernel(src_ref, out_ref):
    g = pl.program_id(0)
    lane = jax.lax.broadcasted_iota(jnp.int32, (32, 2048), 1)
    for u in range(_APB):
        a = _APB * g + u
        limit = 16 * a + 16
        shift = limit & 2047
        rolled = pltpu.roll(src_ref[0:40, :], shift, 1)
        src_rolled = jnp.where(lane < limit, rolled[0:32, :], rolled[1:33, :])
        for h in range(_NUM_HEADS):
            out_ref[h, 16 * u:16 * (u + 1), :] = src_rolled[15 - h:31 - h, :]


def kernel(table, batch, key_len, query_len):
    zero = (
        jnp.asarray(batch, jnp.int32) - 1
        + jnp.asarray(key_len, jnp.int32) - _KEY_LEN
        + jnp.asarray(query_len, jnp.int32) - _QUERY_LEN
    ).reshape(1, 1)
    texp = jnp.tile(table.astype(jnp.float32), (1, _QUERY_LEN // _NUM_HEADS))
    src0 = pl.pallas_call(
        _build_src_kernel,
        out_shape=jax.ShapeDtypeStruct((40, 2048), jnp.float32),
        in_specs=[
            pl.BlockSpec(memory_space=pltpu.SMEM),
            pl.BlockSpec((_NUM_BUCKETS, 2048), lambda: (0, 0)),
        ],
        out_specs=pl.BlockSpec((40, 2048), lambda: (0, 0)),
    )(zero, texp)
    out = pl.pallas_call(
        _expand_kernel,
        grid=(128 // _APB,),
        out_shape=jax.ShapeDtypeStruct((_NUM_HEADS, _KEY_LEN, _QUERY_LEN), jnp.float32),
        in_specs=[pl.BlockSpec((40, 2048), lambda g: (0, 0))],
        out_specs=pl.BlockSpec((_NUM_HEADS, 16 * _APB, _QUERY_LEN), lambda g: (0, g, 0)),
    )(src0)
    return out.reshape(1, _NUM_HEADS, _KEY_LEN, _QUERY_LEN)


# speedup vs baseline: 1.0403x; 1.0403x over previous
import math

import jax
import jax.numpy as jnp
from jax.experimental import pallas as pl
from jax.experimental.pallas import tpu as pltpu

_NUM_HEADS = 16
_NUM_BUCKETS = 32
_MAX_DISTANCE = 128
_KEY_LEN = 2048
_QUERY_LEN = 2048

_APB = 4


def _fused_kernel(zero_ref, texp_ref, out_ref, src_ref):
    g = pl.program_id(0)

    @pl.when(g == 0)
    def _build():
        w = jax.lax.broadcasted_iota(jnp.int32, (40, 2048), 0)
        c = jax.lax.broadcasted_iota(jnp.int32, (40, 2048), 1)
        i_row = w * 128 + (c >> 4)
        d = zero_ref[0, 0] + 2047 - i_row
        nb = _NUM_BUCKETS // 2
        ret = jnp.where(d > 0, nb, 0).astype(jnp.int32)
        ad = jnp.abs(d)
        max_exact = nb // 2
        is_small = ad < max_exact
        rp_f = jnp.maximum(ad, 1).astype(jnp.float32)
        val_large = max_exact + (
            jnp.log(rp_f / max_exact) / math.log(_MAX_DISTANCE / max_exact) * (nb - max_exact)
        ).astype(jnp.int32)
        val_large = jnp.minimum(val_large, nb - 1)
        bucket = ret + jnp.where(is_small, ad, val_large)
        acc = jnp.zeros((40, 2048), jnp.float32)
        for b in range(_NUM_BUCKETS):
            acc = acc + jnp.where(bucket == b, texp_ref[b, :][None, :], 0.0)
        src_ref[:, :] = acc

    lane = jax.lax.broadcasted_iota(jnp.int32, (32, 2048), 1)
    for u in range(_APB):
        a = _APB * g + u
        limit = 16 * a + 16
        shift = limit & 2047
        rolled = pltpu.roll(src_ref[0:40, :], shift, 1)
        src_rolled = jnp.where(lane < limit, rolled[0:32, :], rolled[1:33, :])
        for h in range(_NUM_HEADS):
            out_ref[h, 16 * u:16 * (u + 1), :] = src_rolled[15 - h:31 - h, :]


def kernel(table, batch, key_len, query_len):
    zero = (
        jnp.asarray(batch, jnp.int32) - 1
        + jnp.asarray(key_len, jnp.int32) - _KEY_LEN
        + jnp.asarray(query_len, jnp.int32) - _QUERY_LEN
    ).reshape(1, 1)
    texp = jnp.tile(table.astype(jnp.float32), (1, _QUERY_LEN // _NUM_HEADS))
    out = pl.pallas_call(
        _fused_kernel,
        grid=(128 // _APB,),
        out_shape=jax.ShapeDtypeStruct((_NUM_HEADS, _KEY_LEN, _QUERY_LEN), jnp.float32),
        in_specs=[
            pl.BlockSpec(memory_space=pltpu.SMEM),
            pl.BlockSpec((_NUM_BUCKETS, 2048), lambda g: (0, 0)),
        ],
        out_specs=pl.BlockSpec((_NUM_HEADS, 16 * _APB, _QUERY_LEN), lambda g: (0, g, 0)),
        scratch_shapes=[pltpu.VMEM((40, 2048), jnp.float32)],
    )(zero, texp)
    return out.reshape(1, _NUM_HEADS, _KEY_LEN, _QUERY_LEN)
